# baseline (speedup 1.0000x reference)
import math

import jax
import jax.numpy as jnp
import numpy as np
from jax.experimental import pallas as pl
from jax.experimental.pallas import tpu as pltpu

N = 65536
R = 8
B = 16
LANES = 128
FLAT_ROWS = N * R // LANES
PHS_ROWS = N // LANES
GRID = 16
FB = FLAT_ROWS // GRID
PB = PHS_ROWS // GRID
CH = N // GRID

REF_OSC = np.float32((2.0 * math.pi * 40.0 * 0.001) % (2.0 * math.pi))
STEP = np.float32((math.pi / 4.0) / (N - 1))
C2PI = np.float32(2.0 * math.pi)

_LO = float(REF_OSC)
_HI = float(REF_OSC) + math.pi / 4.0
CENTER = np.float32((_LO + _HI) / 2.0)
CC = np.float32(math.cos((_LO + _HI) / 2.0))
SC = np.float32(math.sin((_LO + _HI) / 2.0))

E01 = ((np.arange(128)[None, :] // 8) == np.arange(16)[:, None]).astype(np.float32)

_lin8 = np.linspace(0.0, 2.0 * math.pi, 8)
COS8 = np.cos(_lin8)[:, None].astype(np.float32)
SIN8 = np.sin(_lin8)[:, None].astype(np.float32)


def _body(spk16_ref, spkd_ref, spkr_ref, e01_ref, cos8_ref, sin8_ref,
          pe_ref, pbt_ref, phs_ref, lsp_ref, pwt_ref):
    g = pl.program_id(0)

    rows_d = jax.lax.broadcasted_iota(jnp.int32, (PB, LANES), 0)
    lane_d = jax.lax.broadcasted_iota(jnp.int32, (PB, LANES), 1)
    i_d = ((g * PB + rows_d) * LANES + lane_d).astype(jnp.float32)
    phid = REF_OSC + i_d * STEP
    phs_ref[...] = phid
    maskd = spkd_ref[...] > 0
    lsp_ref[...] = jnp.where(maskd, phid, -jnp.inf)

    rows_f = jax.lax.broadcasted_iota(jnp.int32, (FB, LANES), 0)
    lane_f = jax.lax.broadcasted_iota(jnp.int32, (FB, LANES), 1)
    i_f = ((g * FB + rows_f) * (LANES // R) + (lane_f >> 3)).astype(jnp.float32)
    k_f = (lane_f & 7).astype(jnp.float32)
    phif = REF_OSC + i_f * STEP
    binf = jnp.floor(phif / C2PI * np.float32(R))
    m16 = (spk16_ref[...] > 0).astype(jnp.float32)
    mrep = jax.lax.dot_general(
        m16, e01_ref[...], (((1,), (0,)), ((), ())),
        preferred_element_type=jnp.float32)
    flat = (np.float32(0.95) * mrep) * (binf == k_f).astype(jnp.float32)
    pe_ref[...] = jnp.broadcast_to(flat[None, :, None, :], (2, FB, R, LANES))

    i_t = (g * CH
           + jax.lax.broadcasted_iota(jnp.int32, (R, CH), 1)).astype(jnp.float32)
    k_t = jax.lax.broadcasted_iota(jnp.int32, (R, CH), 0).astype(jnp.float32)
    phit = REF_OSC + i_t * STEP
    bint = jnp.floor(phit / C2PI * np.float32(R))
    maskt = jnp.broadcast_to(spkr_ref[...] > 0, (R, CH))
    pbt_ref[...] = jnp.where(maskt & (bint == k_t), np.float32(0.95),
                             np.float32(0.0))
    x = phit - CENTER
    x2 = x * x
    cosx = 1.0 + x2 * (np.float32(-0.5) + x2 * (np.float32(1.0 / 24.0)
                                                + x2 * np.float32(-1.0 / 720.0)))
    sinx = x * (1.0 + x2 * (np.float32(-1.0 / 6.0) + x2 * np.float32(1.0 / 120.0)))
    cphi = CC * cosx - SC * sinx
    sphi = SC * cosx + CC * sinx
    c8 = jnp.broadcast_to(cos8_ref[...], (R, CH))
    s8 = jnp.broadcast_to(sin8_ref[...], (R, CH))
    pwt_ref[...] = cphi * c8 + sphi * s8


def _run(spk16, spkd, spkr, e01, cos8, sin8):
    return pl.pallas_call(
        _body,
        grid=(GRID,),
        in_specs=[
            pl.BlockSpec((FB, 16), lambda g: (g, 0)),
            pl.BlockSpec((PB, LANES), lambda g: (g, 0)),
            pl.BlockSpec((1, CH), lambda g: (0, g)),
            pl.BlockSpec((16, LANES), lambda g: (0, 0)),
            pl.BlockSpec((R, 1), lambda g: (0, 0)),
            pl.BlockSpec((R, 1), lambda g: (0, 0)),
        ],
        out_specs=[
            pl.BlockSpec((2, FB, R, LANES), lambda g: (0, g, 0, 0)),
            pl.BlockSpec((R, CH), lambda g: (0, g)),
            pl.BlockSpec((PB, LANES), lambda g: (g, 0)),
            pl.BlockSpec((PB, LANES), lambda g: (g, 0)),
            pl.BlockSpec((R, CH), lambda g: (0, g)),
        ],
        out_shape=[
            jax.ShapeDtypeStruct((2, FLAT_ROWS, R, LANES), jnp.float32),
            jax.ShapeDtypeStruct((R, N), jnp.float32),
            jax.ShapeDtypeStruct((PHS_ROWS, LANES), jnp.float32),
            jax.ShapeDtypeStruct((PHS_ROWS, LANES), jnp.float32),
            jax.ShapeDtypeStruct((R, N), jnp.float32),
        ],
        compiler_params=pltpu.CompilerParams(
            dimension_semantics=("parallel",)),
    )(spk16, spkd, spkr, e01, cos8, sin8)


def kernel(input_spikes, current_time):
    row0 = input_spikes[0]
    spk16 = row0.reshape(FLAT_ROWS, 16)
    spkd = row0.reshape(PHS_ROWS, LANES)
    spkr = row0.reshape(1, N)
    pe4, pbt, phs, lsp, pwt = _run(
        spk16, spkd, spkr, jnp.asarray(E01), jnp.asarray(COS8), jnp.asarray(SIN8))
    phase_encoded = jnp.transpose(pe4, (0, 2, 1, 3)).reshape(B, N * R)
    current_phases = phs.reshape(N)
    phase_bins = pbt.T
    reference_phase = jnp.asarray(REF_OSC, dtype=jnp.float32)
    last_spike_phases = lsp.reshape(N)
    phase_weights = pwt.T
    return (phase_encoded, current_phases, phase_bins, reference_phase,
            last_spike_phases, phase_weights)


# speedup vs baseline: 1.1728x; 1.1728x over previous
import math

import jax
import jax.numpy as jnp
import numpy as np
from jax.experimental import pallas as pl
from jax.experimental.pallas import tpu as pltpu

N = 65536
R = 8
B = 16
LANES = 128
FLAT_ROWS = N * R // LANES
PHS_ROWS = N // LANES
GRID = 4
FB = FLAT_ROWS // GRID
PB = PHS_ROWS // GRID
CH = N // GRID

REF_OSC = np.float32((2.0 * math.pi * 40.0 * 0.001) % (2.0 * math.pi))
STEP = np.float32((math.pi / 4.0) / (N - 1))
C2PI = np.float32(2.0 * math.pi)

_LO = float(REF_OSC)
_HI = float(REF_OSC) + math.pi / 4.0
CENTER = np.float32((_LO + _HI) / 2.0)
CC = np.float32(math.cos((_LO + _HI) / 2.0))
SC = np.float32(math.sin((_LO + _HI) / 2.0))

E01 = ((np.arange(128)[None, :] // 8) == np.arange(16)[:, None]).astype(np.float32)

_lin8 = np.linspace(0.0, 2.0 * math.pi, 8)
COS8 = np.cos(_lin8)[:, None].astype(np.float32)
SIN8 = np.sin(_lin8)[:, None].astype(np.float32)


def _body(spk16_ref, spkd_ref, spkr_ref, e01_ref, cos8_ref, sin8_ref,
          pe_ref, pbt_ref, phs_ref, lsp_ref, pwt_ref):
    g = pl.program_id(0)

    rows_d = jax.lax.broadcasted_iota(jnp.int32, (PB, LANES), 0)
    lane_d = jax.lax.broadcasted_iota(jnp.int32, (PB, LANES), 1)
    i_d = ((g * PB + rows_d) * LANES + lane_d).astype(jnp.float32)
    phid = REF_OSC + i_d * STEP
    phs_ref[...] = phid
    maskd = spkd_ref[...] > 0
    lsp_ref[...] = jnp.where(maskd, phid, -jnp.inf)

    rows_f = jax.lax.broadcasted_iota(jnp.int32, (FB, LANES), 0)
    lane_f = jax.lax.broadcasted_iota(jnp.int32, (FB, LANES), 1)
    i_f = ((g * FB + rows_f) * (LANES // R) + (lane_f >> 3)).astype(jnp.float32)
    k_f = (lane_f & 7).astype(jnp.float32)
    phif = REF_OSC + i_f * STEP
    binf = jnp.floor(phif / C2PI * np.float32(R))
    m16 = (spk16_ref[...] > 0).astype(jnp.float32)
    mrep = jax.lax.dot_general(
        m16, e01_ref[...], (((1,), (0,)), ((), ())),
        preferred_element_type=jnp.float32)
    flat = (np.float32(0.95) * mrep) * (binf == k_f).astype(jnp.float32)
    pe_ref[...] = jnp.broadcast_to(flat[None, :, None, :], (2, FB, R, LANES))

    i_t = (g * CH
           + jax.lax.broadcasted_iota(jnp.int32, (R, CH), 1)).astype(jnp.float32)
    k_t = jax.lax.broadcasted_iota(jnp.int32, (R, CH), 0).astype(jnp.float32)
    phit = REF_OSC + i_t * STEP
    bint = jnp.floor(phit / C2PI * np.float32(R))
    maskt = jnp.broadcast_to(spkr_ref[...] > 0, (R, CH))
    pbt_ref[...] = jnp.where(maskt & (bint == k_t), np.float32(0.95),
                             np.float32(0.0))
    x = phit - CENTER
    x2 = x * x
    cosx = 1.0 + x2 * (np.float32(-0.5) + x2 * (np.float32(1.0 / 24.0)
                                                + x2 * np.float32(-1.0 / 720.0)))
    sinx = x * (1.0 + x2 * (np.float32(-1.0 / 6.0) + x2 * np.float32(1.0 / 120.0)))
    cphi = CC * cosx - SC * sinx
    sphi = SC * cosx + CC * sinx
    c8 = jnp.broadcast_to(cos8_ref[...], (R, CH))
    s8 = jnp.broadcast_to(sin8_ref[...], (R, CH))
    pwt_ref[...] = cphi * c8 + sphi * s8


def _run(spk16, spkd, spkr, e01, cos8, sin8):
    return pl.pallas_call(
        _body,
        grid=(GRID,),
        in_specs=[
            pl.BlockSpec((FB, 16), lambda g: (g, 0)),
            pl.BlockSpec((PB, LANES), lambda g: (g, 0)),
            pl.BlockSpec((1, CH), lambda g: (0, g)),
            pl.BlockSpec((16, LANES), lambda g: (0, 0)),
            pl.BlockSpec((R, 1), lambda g: (0, 0)),
            pl.BlockSpec((R, 1), lambda g: (0, 0)),
        ],
        out_specs=[
            pl.BlockSpec((2, FB, R, LANES), lambda g: (0, g, 0, 0)),
            pl.BlockSpec((R, CH), lambda g: (0, g)),
            pl.BlockSpec((PB, LANES), lambda g: (g, 0)),
            pl.BlockSpec((PB, LANES), lambda g: (g, 0)),
            pl.BlockSpec((R, CH), lambda g: (0, g)),
        ],
        out_shape=[
            jax.ShapeDtypeStruct((2, FLAT_ROWS, R, LANES), jnp.float32),
            jax.ShapeDtypeStruct((R, N), jnp.float32),
            jax.ShapeDtypeStruct((PHS_ROWS, LANES), jnp.float32),
            jax.ShapeDtypeStruct((PHS_ROWS, LANES), jnp.float32),
            jax.ShapeDtypeStruct((R, N), jnp.float32),
        ],
        compiler_params=pltpu.CompilerParams(
            dimension_semantics=("parallel",)),
    )(spk16, spkd, spkr, e01, cos8, sin8)


def kernel(input_spikes, current_time):
    row0 = input_spikes[0]
    spk16 = row0.reshape(FLAT_ROWS, 16)
    spkd = row0.reshape(PHS_ROWS, LANES)
    spkr = row0.reshape(1, N)
    pe4, pbt, phs, lsp, pwt = _run(
        spk16, spkd, spkr, jnp.asarray(E01), jnp.asarray(COS8), jnp.asarray(SIN8))
    phase_encoded = jnp.transpose(pe4, (0, 2, 1, 3)).reshape(B, N * R)
    current_phases = phs.reshape(N)
    phase_bins = pbt.T
    reference_phase = jnp.asarray(REF_OSC, dtype=jnp.float32)
    last_spike_phases = lsp.reshape(N)
    phase_weights = pwt.T
    return (phase_encoded, current_phases, phase_bins, reference_phase,
            last_spike_phases, phase_weights)
